# Initial kernel scaffold; baseline (speedup 1.0000x reference)
import jax
import jax.numpy as jnp
from jax.experimental import pallas as pl


def kernel(x, edge_index, rel_type, norm, weight):
    raise NotImplementedError("write your pallas kernel here")


# speedup vs baseline: 10.6558x; 10.6558x over previous
import functools

import jax
import jax.numpy as jnp
from jax import lax
from jax.experimental import pallas as pl
from jax.experimental.pallas import tpu as pltpu
from jax.experimental.pallas import tpu_sc as plsc

N = 10000
E = 320000
D = 128
R = 8

NC = 2
NS = 16
NW = NC * NS
LANES = 16

CHUNK = 128
EPW_CHUNKS = -(-E // (NW * CHUNK))
E_PAD = NW * CHUNK * EPW_CHUNKS

N_PAD = 10240
ZROWS = 128
OROWS = N_PAD // NS


def _matmul_body(x_ref, w_ref, o_ref):
    o_ref[...] = jnp.dot(x_ref[...], w_ref[0],
                         preferred_element_type=jnp.float32)


def _transform(x, weight):
    bn = 2000
    nb = N // bn
    return pl.pallas_call(
        _matmul_body,
        grid=(R, nb),
        in_specs=[
            pl.BlockSpec((bn, D), lambda r, n: (n, 0)),
            pl.BlockSpec((1, D, D), lambda r, n: (r, 0, 0)),
        ],
        out_specs=pl.BlockSpec((bn, D), lambda r, n: (r * nb + n, 0)),
        out_shape=jax.ShapeDtypeStruct((R * N, D), jnp.float32),
    )(x, weight)


def _sc_edge_body(table, idx, dst, norm, out,
                  idx_v, dst_v, norm_v, rows_v, acc, sem):
    cid = lax.axis_index("c")
    sid = lax.axis_index("s")
    wid = sid * NC + cid

    zero16 = jnp.zeros((LANES,), jnp.float32)

    def _zero_row(e, _):
        for j in range(D // LANES):
            rows_v[e, pl.ds(j * LANES, LANES)] = zero16
        return _

    lax.fori_loop(0, ZROWS, _zero_row, None)
    for k in range(640 // ZROWS):
        pltpu.sync_copy(rows_v.at[pl.ds(0, ZROWS)],
                        acc.at[pl.ds(sid * 640 + k * ZROWS, ZROWS)])
    plsc.subcore_barrier()

    def _chunk(c, _):
        base = (wid * EPW_CHUNKS + c) * CHUNK
        pltpu.sync_copy(idx.at[pl.ds(base, CHUNK)], idx_v)
        pltpu.sync_copy(dst.at[pl.ds(base, CHUNK)], dst_v)
        pltpu.sync_copy(norm.at[pl.ds(base, CHUNK)], norm_v)
        pltpu.async_copy(table.at[idx_v], rows_v, sem).wait()

        def _scale(g, _):
            nv16 = norm_v[pl.ds(g * LANES, LANES)]
            for e in range(LANES):
                row = g * LANES + e
                nv = jnp.full((LANES,), nv16[e], jnp.float32)
                for j in range(D // LANES):
                    sl = pl.ds(j * LANES, LANES)
                    rows_v[row, sl] = rows_v[row, sl] * nv
            return _

        lax.fori_loop(0, CHUNK // LANES, _scale, None)
        pltpu.sync_copy(rows_v, acc.at[dst_v], add=True)
        return _

    lax.fori_loop(0, EPW_CHUNKS, _chunk, None)
    plsc.subcore_barrier()

    pltpu.sync_copy(acc.at[pl.ds(sid * OROWS, OROWS)],
                    out.at[pl.ds(cid * N_PAD + sid * OROWS, OROWS)])


@functools.partial(
    pl.kernel,
    mesh=plsc.VectorSubcoreMesh(core_axis_name="c", subcore_axis_name="s",
                                num_cores=NC, num_subcores=NS),
    out_type=jax.ShapeDtypeStruct((NC * N_PAD, D), jnp.float32),
    scratch_types=[
        pltpu.VMEM((CHUNK,), jnp.int32),
        pltpu.VMEM((CHUNK,), jnp.int32),
        pltpu.VMEM((CHUNK,), jnp.float32),
        pltpu.VMEM((CHUNK, D), jnp.float32),
        pltpu.VMEM_SHARED((N_PAD, D), jnp.float32),
        pltpu.SemaphoreType.DMA,
    ],
)
def _sc_edge_kernel(table, idx, dst, norm, out,
                    idx_v, dst_v, norm_v, rows_v, acc, sem):
    _sc_edge_body(table, idx, dst, norm, out,
                  idx_v, dst_v, norm_v, rows_v, acc, sem)


def _add_body(a_ref, b_ref, o_ref):
    o_ref[...] = a_ref[...] + b_ref[...]


def _combine(partials):
    bn = 2000
    return pl.pallas_call(
        _add_body,
        grid=(N // bn,),
        in_specs=[
            pl.BlockSpec((bn, D), lambda i: (i, 0)),
            pl.BlockSpec((bn, D), lambda i: (i, 0)),
        ],
        out_specs=pl.BlockSpec((bn, D), lambda i: (i, 0)),
        out_shape=jax.ShapeDtypeStruct((N, D), jnp.float32),
    )(partials[:N], partials[N_PAD:N_PAD + N])


def kernel(x, edge_index, rel_type, norm, weight):
    src = edge_index[0].astype(jnp.int32)
    dst = edge_index[1].astype(jnp.int32)
    rel = rel_type.astype(jnp.int32)
    idx = rel * N + src

    pad = E_PAD - E
    idx_p = jnp.concatenate([idx, jnp.zeros((pad,), jnp.int32)])
    dst_p = jnp.concatenate([dst, jnp.zeros((pad,), jnp.int32)])
    norm_p = jnp.concatenate([norm[:, 0], jnp.zeros((pad,), jnp.float32)])

    table = _transform(x, weight)
    partials = _sc_edge_kernel(table, idx_p, dst_p, norm_p)
    return _combine(partials)


# speedup vs baseline: 18.0052x; 1.6897x over previous
import functools

import jax
import jax.numpy as jnp
from jax import lax
from jax.experimental import pallas as pl
from jax.experimental.pallas import tpu as pltpu
from jax.experimental.pallas import tpu_sc as plsc

N = 10000
E = 320000
D = 128
R = 8

NC = 2
NS = 16
NW = NC * NS
LANES = 16

CHUNK = 112
NCH = 90
UNROLL = 6
E_PAD = NW * CHUNK * NCH

N_PAD = 10240
OROWS = N_PAD // NS


def _matmul_body(x_ref, w_ref, o_ref):
    o_ref[...] = jnp.dot(x_ref[...], w_ref[0],
                         preferred_element_type=jnp.float32)


def _transform(x, weight):
    bn = 2000
    nb = N // bn
    return pl.pallas_call(
        _matmul_body,
        grid=(R, nb),
        in_specs=[
            pl.BlockSpec((bn, D), lambda r, n: (n, 0)),
            pl.BlockSpec((1, D, D), lambda r, n: (r, 0, 0)),
        ],
        out_specs=pl.BlockSpec((bn, D), lambda r, n: (r * nb + n, 0)),
        out_shape=jax.ShapeDtypeStruct((R * N, D), jnp.float32),
    )(x, weight)


def _sc_edge_body(table, idx, dst, norm, out,
                  idx3, dst3, nm3, rows2, acc, sem_s, sem_g, sem_sc):
    cid = lax.axis_index("c")
    sid = lax.axis_index("s")
    wid = sid * NC + cid

    zero16 = jnp.zeros((LANES,), jnp.float32)

    def _zero_row(e, _):
        for j in range(D // LANES):
            rows2[0, e, pl.ds(j * LANES, LANES)] = zero16
        return _

    lax.fori_loop(0, CHUNK, _zero_row, None)
    zoff = 0
    for zr in (CHUNK, CHUNK, CHUNK, CHUNK, CHUNK, OROWS - 5 * CHUNK):
        pltpu.sync_copy(rows2.at[0, pl.ds(0, zr)],
                        acc.at[pl.ds(sid * OROWS + zoff, zr)])
        zoff += zr
    plsc.subcore_barrier()

    def _small(c, s):
        base = (wid * NCH + c) * CHUNK
        pltpu.async_copy(idx.at[pl.ds(base, CHUNK)], idx3.at[s], sem_s[s])
        pltpu.async_copy(dst.at[pl.ds(base, CHUNK)], dst3.at[s], sem_s[s])
        pltpu.async_copy(norm.at[pl.ds(base, CHUNK)], nm3.at[s], sem_s[s])

    def _wait_small(c, s):
        base = (wid * NCH + c) * CHUNK
        pltpu.make_async_copy(idx.at[pl.ds(base, CHUNK)], idx3.at[s],
                              sem_s[s]).wait()
        pltpu.make_async_copy(dst.at[pl.ds(base, CHUNK)], dst3.at[s],
                              sem_s[s]).wait()
        pltpu.make_async_copy(norm.at[pl.ds(base, CHUNK)], nm3.at[s],
                              sem_s[s]).wait()

    def _gather(s, rb):
        pltpu.async_copy(table.at[idx3.at[s]], rows2.at[rb], sem_g[rb])

    def _wait_gather(s, rb):
        pltpu.make_async_copy(table.at[idx3.at[s]], rows2.at[rb],
                              sem_g[rb]).wait()

    def _scatter(s, rb):
        pltpu.async_copy(rows2.at[rb], acc.at[dst3.at[s]], sem_sc[rb],
                         add=True)

    def _wait_scatter(s, rb):
        pltpu.make_async_copy(rows2.at[rb], acc.at[dst3.at[s]],
                              sem_sc[rb]).wait()

    _small(0, 0)
    _small(1, 1)
    _wait_small(0, 0)
    _gather(0, 0)

    nblk = NCH // UNROLL

    def _outer(i, _):
        for k in range(UNROLL):
            c = i * UNROLL + k
            s_cur = k % 3
            s_nxt = (k + 1) % 3
            s_n2 = (k + 2) % 3
            rb = k % 2
            last_blk = i == nblk - 1

            if k == UNROLL - 1:
                @pl.when(jnp.logical_not(last_blk))
                def _():
                    _wait_small(c + 1, s_nxt)
                    _wait_scatter((k - 1) % 3, 1 - rb)
                    _gather(s_nxt, 1 - rb)
                    _small(c + 2, s_n2)

                @pl.when(last_blk)
                def _():
                    _wait_scatter((k - 1) % 3, 1 - rb)
            else:
                _wait_small(c + 1, s_nxt)
                if k == 0:
                    @pl.when(i > 0)
                    def _():
                        _wait_scatter((k - 1) % 3, 1 - rb)
                else:
                    _wait_scatter((k - 1) % 3, 1 - rb)
                _gather(s_nxt, 1 - rb)
                if k == UNROLL - 2:
                    @pl.when(jnp.logical_not(last_blk))
                    def _():
                        _small(c + 2, s_n2)
                else:
                    _small(c + 2, s_n2)

            _wait_gather(s_cur, rb)

            def _scale(g, carry):
                nv16 = nm3[s_cur, pl.ds(g * LANES, LANES)]
                for e in range(LANES):
                    row = g * LANES + e
                    nv = jnp.full((LANES,), nv16[e], jnp.float32)
                    for j in range(D // LANES):
                        sl = pl.ds(j * LANES, LANES)
                        rows2[rb, row, sl] = rows2[rb, row, sl] * nv
                return carry

            lax.fori_loop(0, CHUNK // LANES, _scale, None)
            _scatter(s_cur, rb)
        return _

    lax.fori_loop(0, nblk, _outer, None)
    _wait_scatter((NCH - 1) % 3, (NCH - 1) % 2)
    plsc.subcore_barrier()

    pltpu.sync_copy(acc.at[pl.ds(sid * OROWS, OROWS)],
                    out.at[pl.ds(cid * N_PAD + sid * OROWS, OROWS)])


@functools.partial(
    pl.kernel,
    mesh=plsc.VectorSubcoreMesh(core_axis_name="c", subcore_axis_name="s",
                                num_cores=NC, num_subcores=NS),
    out_type=jax.ShapeDtypeStruct((NC * N_PAD, D), jnp.float32),
    scratch_types=[
        pltpu.VMEM((3, CHUNK), jnp.int32),
        pltpu.VMEM((3, CHUNK), jnp.int32),
        pltpu.VMEM((3, CHUNK), jnp.float32),
        pltpu.VMEM((2, CHUNK, D), jnp.float32),
        pltpu.VMEM_SHARED((N_PAD, D), jnp.float32),
        [pltpu.SemaphoreType.DMA] * 3,
        [pltpu.SemaphoreType.DMA] * 2,
        [pltpu.SemaphoreType.DMA] * 2,
    ],
)
def _sc_edge_kernel(table, idx, dst, norm, out,
                    idx3, dst3, nm3, rows2, acc, sem_s, sem_g, sem_sc):
    _sc_edge_body(table, idx, dst, norm, out,
                  idx3, dst3, nm3, rows2, acc, sem_s, sem_g, sem_sc)


def _add_body(a_ref, b_ref, o_ref):
    o_ref[...] = a_ref[...] + b_ref[...]


def _combine(partials):
    bn = 2000
    return pl.pallas_call(
        _add_body,
        grid=(N // bn,),
        in_specs=[
            pl.BlockSpec((bn, D), lambda i: (i, 0)),
            pl.BlockSpec((bn, D), lambda i: (i, 0)),
        ],
        out_specs=pl.BlockSpec((bn, D), lambda i: (i, 0)),
        out_shape=jax.ShapeDtypeStruct((N, D), jnp.float32),
    )(partials[:N], partials[N_PAD:N_PAD + N])


def kernel(x, edge_index, rel_type, norm, weight):
    src = edge_index[0].astype(jnp.int32)
    dst = edge_index[1].astype(jnp.int32)
    rel = rel_type.astype(jnp.int32)
    idx = rel * N + src

    pad = E_PAD - E
    idx_p = jnp.concatenate([idx, jnp.zeros((pad,), jnp.int32)])
    dst_p = jnp.concatenate([dst, jnp.zeros((pad,), jnp.int32)])
    norm_p = jnp.concatenate([norm[:, 0], jnp.zeros((pad,), jnp.float32)])

    table = _transform(x, weight)
    partials = _sc_edge_kernel(table, idx_p, dst_p, norm_p)
    return _combine(partials)


# speedup vs baseline: 21.5546x; 1.1971x over previous
import functools

import jax
import jax.numpy as jnp
from jax import lax
from jax.experimental import pallas as pl
from jax.experimental.pallas import tpu as pltpu
from jax.experimental.pallas import tpu_sc as plsc

N = 10000
E = 320000
D = 128
R = 8

NC = 2
NS = 16
NW = NC * NS
LANES = 16

CHUNK = 112
NCH = 90
NCH0 = 114
NCH1 = 2 * NCH - NCH0
UNROLL = 6
E_PAD = NW * CHUNK * NCH

N_PAD = 10240
OROWS = N_PAD // NS


def _matmul_body(x_ref, w_ref, o_ref):
    xb = x_ref[...]
    for r in range(R):
        o_ref[r] = jnp.dot(xb, w_ref[r], preferred_element_type=jnp.float32)


def _transform(x, weight):
    bn = 2000
    nb = N // bn
    out = pl.pallas_call(
        _matmul_body,
        grid=(nb,),
        in_specs=[
            pl.BlockSpec((bn, D), lambda n: (n, 0)),
            pl.BlockSpec((R, D, D), lambda n: (0, 0, 0)),
        ],
        out_specs=pl.BlockSpec((R, bn, D), lambda n: (0, n, 0)),
        out_shape=jax.ShapeDtypeStruct((R, N, D), jnp.float32),
    )(x, weight)
    return out.reshape(R * N, D)


def _sc_edge_body(table, idx, dst, norm, out,
                  idx3, dst3, nm3, rows3, acc, sem_s, sem_g, sem_sc):
    cid = lax.axis_index("c")
    sid = lax.axis_index("s")
    cbase = sid * (2 * NCH) + jnp.where(cid == 0, 0, NCH0)
    my_nch = jnp.where(cid == 0, NCH0, NCH1)

    zero16 = jnp.zeros((LANES,), jnp.float32)

    def _zero_row(e, _):
        for j in range(D // LANES):
            rows3[0, e, pl.ds(j * LANES, LANES)] = zero16
        return _

    lax.fori_loop(0, CHUNK, _zero_row, None)
    zoff = 0
    for zr in (CHUNK, CHUNK, CHUNK, CHUNK, CHUNK, OROWS - 5 * CHUNK):
        pltpu.sync_copy(rows3.at[0, pl.ds(0, zr)],
                        acc.at[pl.ds(sid * OROWS + zoff, zr)])
        zoff += zr
    plsc.subcore_barrier()

    def _small(c, s):
        base = (cbase + c) * CHUNK
        pltpu.async_copy(idx.at[pl.ds(base, CHUNK)], idx3.at[s], sem_s[s])
        pltpu.async_copy(dst.at[pl.ds(base, CHUNK)], dst3.at[s], sem_s[s])
        pltpu.async_copy(norm.at[pl.ds(base, CHUNK)], nm3.at[s], sem_s[s])

    def _wait_small(c, s):
        base = (cbase + c) * CHUNK
        pltpu.make_async_copy(idx.at[pl.ds(base, CHUNK)], idx3.at[s],
                              sem_s[s]).wait()
        pltpu.make_async_copy(dst.at[pl.ds(base, CHUNK)], dst3.at[s],
                              sem_s[s]).wait()
        pltpu.make_async_copy(norm.at[pl.ds(base, CHUNK)], nm3.at[s],
                              sem_s[s]).wait()

    def _gather(s, rb):
        pltpu.async_copy(table.at[idx3.at[s]], rows3.at[rb], sem_g[rb])

    def _wait_gather(s, rb):
        pltpu.make_async_copy(table.at[idx3.at[s]], rows3.at[rb],
                              sem_g[rb]).wait()

    def _scatter(s, rb):
        pltpu.async_copy(rows3.at[rb], acc.at[dst3.at[s]], sem_sc[rb],
                         add=True)

    def _wait_scatter(s, rb):
        pltpu.make_async_copy(rows3.at[rb], acc.at[dst3.at[s]],
                              sem_sc[rb]).wait()

    _small(0, 0)
    _small(1, 1)
    _wait_small(0, 0)
    _gather(0, 0)

    nblk = my_nch // UNROLL

    def _outer(i, _):
        for k in range(UNROLL):
            c = i * UNROLL + k
            s_cur = k % 6
            s_nxt = (k + 1) % 6
            s_n2 = (k + 2) % 6
            rb = k % 3
            rb_nxt = (k + 1) % 3
            last_blk = i == nblk - 1

            def _steady(pred_sc):
                if pred_sc is None:
                    _wait_scatter((k - 2) % 6, rb_nxt)
                else:
                    @pl.when(pred_sc)
                    def _():
                        _wait_scatter((k - 2) % 6, rb_nxt)

            if k == UNROLL - 1:
                @pl.when(jnp.logical_not(last_blk))
                def _():
                    _wait_small(c + 1, s_nxt)
                    _wait_scatter((k - 2) % 6, rb_nxt)
                    _gather(s_nxt, rb_nxt)
                    _small(c + 2, s_n2)

                @pl.when(last_blk)
                def _():
                    _wait_scatter((k - 2) % 6, rb_nxt)
            else:
                _wait_small(c + 1, s_nxt)
                _steady(i > 0 if k < 2 else None)
                _gather(s_nxt, rb_nxt)
                if k == UNROLL - 2:
                    @pl.when(jnp.logical_not(last_blk))
                    def _():
                        _small(c + 2, s_n2)
                else:
                    _small(c + 2, s_n2)

            _wait_gather(s_cur, rb)

            def _scale(g, carry):
                nv16 = nm3[s_cur, pl.ds(g * LANES, LANES)]
                for e in range(LANES):
                    row = g * LANES + e
                    nv = jnp.full((LANES,), nv16[e], jnp.float32)
                    for j in range(D // LANES):
                        sl = pl.ds(j * LANES, LANES)
                        rows3[rb, row, sl] = rows3[rb, row, sl] * nv
                return carry

            lax.fori_loop(0, CHUNK // LANES, _scale, None)
            _scatter(s_cur, rb)
        return _

    lax.fori_loop(0, nblk, _outer, None)
    _wait_scatter(4, 1)
    _wait_scatter(5, 2)
    plsc.subcore_barrier()

    pltpu.sync_copy(acc.at[pl.ds(sid * OROWS, OROWS)],
                    out.at[pl.ds(cid * N_PAD + sid * OROWS, OROWS)])


@functools.partial(
    pl.kernel,
    mesh=plsc.VectorSubcoreMesh(core_axis_name="c", subcore_axis_name="s",
                                num_cores=NC, num_subcores=NS),
    out_type=jax.ShapeDtypeStruct((NC * N_PAD, D), jnp.float32),
    scratch_types=[
        pltpu.VMEM((6, CHUNK), jnp.int32),
        pltpu.VMEM((6, CHUNK), jnp.int32),
        pltpu.VMEM((6, CHUNK), jnp.float32),
        pltpu.VMEM((3, CHUNK, D), jnp.float32),
        pltpu.VMEM_SHARED((N_PAD, D), jnp.float32),
        [pltpu.SemaphoreType.DMA] * 6,
        [pltpu.SemaphoreType.DMA] * 3,
        [pltpu.SemaphoreType.DMA] * 3,
    ],
)
def _sc_edge_kernel(table, idx, dst, norm, out,
                    idx3, dst3, nm3, rows3, acc, sem_s, sem_g, sem_sc):
    _sc_edge_body(table, idx, dst, norm, out,
                  idx3, dst3, nm3, rows3, acc, sem_s, sem_g, sem_sc)


def _add_body(a_ref, b_ref, o_ref):
    o_ref[...] = a_ref[...] + b_ref[...]


def _combine(partials):
    bn = 2000
    return pl.pallas_call(
        _add_body,
        grid=(N // bn,),
        in_specs=[
            pl.BlockSpec((bn, D), lambda i: (i, 0)),
            pl.BlockSpec((bn, D), lambda i: (i, 0)),
        ],
        out_specs=pl.BlockSpec((bn, D), lambda i: (i, 0)),
        out_shape=jax.ShapeDtypeStruct((N, D), jnp.float32),
    )(partials[:N], partials[N_PAD:N_PAD + N])


def kernel(x, edge_index, rel_type, norm, weight):
    src = edge_index[0].astype(jnp.int32)
    dst = edge_index[1].astype(jnp.int32)
    rel = rel_type.astype(jnp.int32)
    idx = rel * N + src

    pad = E_PAD - E
    idx_p = jnp.concatenate([idx, jnp.zeros((pad,), jnp.int32)])
    dst_p = jnp.concatenate([dst, jnp.zeros((pad,), jnp.int32)])
    norm_p = jnp.concatenate([norm[:, 0], jnp.zeros((pad,), jnp.float32)])

    table = _transform(x, weight)
    partials = _sc_edge_kernel(table, idx_p, dst_p, norm_p)
    return _combine(partials)


# speedup vs baseline: 22.3135x; 1.0352x over previous
import functools

import jax
import jax.numpy as jnp
from jax import lax
from jax.experimental import pallas as pl
from jax.experimental.pallas import tpu as pltpu
from jax.experimental.pallas import tpu_sc as plsc

N = 10000
E = 320000
D = 128
R = 8

NC = 2
NS = 16
NW = NC * NS
LANES = 16

CHUNK = 112
NCH = 90
NCH0 = 126
NCH1 = 2 * NCH - NCH0
UNROLL = 6
E_PAD = NW * CHUNK * NCH

N_PAD = 10240
OROWS = N_PAD // NS


def _matmul_body(x_ref, w_ref, o_ref):
    xb = x_ref[...]
    for r in range(R):
        o_ref[r] = jnp.dot(xb, w_ref[r], preferred_element_type=jnp.float32)


def _transform(x, weight):
    bn = 2000
    nb = N // bn
    out = pl.pallas_call(
        _matmul_body,
        grid=(nb,),
        in_specs=[
            pl.BlockSpec((bn, D), lambda n: (n, 0)),
            pl.BlockSpec((R, D, D), lambda n: (0, 0, 0)),
        ],
        out_specs=pl.BlockSpec((R, bn, D), lambda n: (0, n, 0)),
        out_shape=jax.ShapeDtypeStruct((R, N, D), jnp.float32),
    )(x, weight)
    return out.reshape(R * N, D)


def _sc_edge_body(table, idx, dst, norm, out,
                  idx3, dst3, nm3, rows3, acc, sem_s, sem_g, sem_sc):
    cid = lax.axis_index("c")
    sid = lax.axis_index("s")
    cbase = sid * (2 * NCH) + jnp.where(cid == 0, 0, NCH0)
    my_nch = jnp.where(cid == 0, NCH0, NCH1)

    zero16 = jnp.zeros((LANES,), jnp.float32)

    def _zero_row(e, _):
        for j in range(D // LANES):
            rows3[0, e, pl.ds(j * LANES, LANES)] = zero16
        return _

    lax.fori_loop(0, CHUNK, _zero_row, None)
    zoff = 0
    for zr in (CHUNK, CHUNK, CHUNK, CHUNK, CHUNK, OROWS - 5 * CHUNK):
        pltpu.sync_copy(rows3.at[0, pl.ds(0, zr)],
                        acc.at[pl.ds(sid * OROWS + zoff, zr)])
        zoff += zr
    plsc.subcore_barrier()

    def _small(c, s):
        base = (cbase + c) * CHUNK
        pltpu.async_copy(idx.at[pl.ds(base, CHUNK)], idx3.at[s], sem_s[s])
        pltpu.async_copy(dst.at[pl.ds(base, CHUNK)], dst3.at[s], sem_s[s])
        pltpu.async_copy(norm.at[pl.ds(base, CHUNK)], nm3.at[s], sem_s[s])

    def _wait_small(c, s):
        base = (cbase + c) * CHUNK
        pltpu.make_async_copy(idx.at[pl.ds(base, CHUNK)], idx3.at[s],
                              sem_s[s]).wait()
        pltpu.make_async_copy(dst.at[pl.ds(base, CHUNK)], dst3.at[s],
                              sem_s[s]).wait()
        pltpu.make_async_copy(norm.at[pl.ds(base, CHUNK)], nm3.at[s],
                              sem_s[s]).wait()

    def _gather(s, rb):
        pltpu.async_copy(table.at[idx3.at[s]], rows3.at[rb], sem_g[rb])

    def _wait_gather(s, rb):
        pltpu.make_async_copy(table.at[idx3.at[s]], rows3.at[rb],
                              sem_g[rb]).wait()

    def _scatter(s, rb):
        pltpu.async_copy(rows3.at[rb], acc.at[dst3.at[s]], sem_sc[rb],
                         add=True)

    def _wait_scatter(s, rb):
        pltpu.make_async_copy(rows3.at[rb], acc.at[dst3.at[s]],
                              sem_sc[rb]).wait()

    _small(0, 0)
    _small(1, 1)
    _wait_small(0, 0)
    _gather(0, 0)

    nblk = my_nch // UNROLL

    def _outer(i, _):
        for k in range(UNROLL):
            c = i * UNROLL + k
            s_cur = k % 6
            s_nxt = (k + 1) % 6
            s_n2 = (k + 2) % 6
            rb = k % 3
            rb_nxt = (k + 1) % 3
            last_blk = i == nblk - 1

            def _steady(pred_sc):
                if pred_sc is None:
                    _wait_scatter((k - 2) % 6, rb_nxt)
                else:
                    @pl.when(pred_sc)
                    def _():
                        _wait_scatter((k - 2) % 6, rb_nxt)

            if k == UNROLL - 1:
                @pl.when(jnp.logical_not(last_blk))
                def _():
                    _wait_small(c + 1, s_nxt)
                    _wait_scatter((k - 2) % 6, rb_nxt)
                    _gather(s_nxt, rb_nxt)
                    _small(c + 2, s_n2)

                @pl.when(last_blk)
                def _():
                    _wait_scatter((k - 2) % 6, rb_nxt)
            else:
                _wait_small(c + 1, s_nxt)
                _steady(i > 0 if k < 2 else None)
                _gather(s_nxt, rb_nxt)
                if k == UNROLL - 2:
                    @pl.when(jnp.logical_not(last_blk))
                    def _():
                        _small(c + 2, s_n2)
                else:
                    _small(c + 2, s_n2)

            _wait_gather(s_cur, rb)

            def _scale(g, carry):
                nv16 = nm3[s_cur, pl.ds(g * LANES, LANES)]
                for e in range(LANES):
                    row = g * LANES + e
                    nv = jnp.full((LANES,), nv16[e], jnp.float32)
                    for j in range(D // LANES):
                        sl = pl.ds(j * LANES, LANES)
                        rows3[rb, row, sl] = rows3[rb, row, sl] * nv
                return carry

            lax.fori_loop(0, CHUNK // LANES, _scale, None)
            _scatter(s_cur, rb)
        return _

    lax.fori_loop(0, nblk, _outer, None)
    _wait_scatter(4, 1)
    _wait_scatter(5, 2)
    plsc.subcore_barrier()

    pltpu.sync_copy(acc.at[pl.ds(sid * OROWS, OROWS)],
                    out.at[pl.ds(cid * N_PAD + sid * OROWS, OROWS)])


@functools.partial(
    pl.kernel,
    mesh=plsc.VectorSubcoreMesh(core_axis_name="c", subcore_axis_name="s",
                                num_cores=NC, num_subcores=NS),
    out_type=jax.ShapeDtypeStruct((NC * N_PAD, D), jnp.float32),
    scratch_types=[
        pltpu.VMEM((6, CHUNK), jnp.int32),
        pltpu.VMEM((6, CHUNK), jnp.int32),
        pltpu.VMEM((6, CHUNK), jnp.float32),
        pltpu.VMEM((3, CHUNK, D), jnp.float32),
        pltpu.VMEM_SHARED((N_PAD, D), jnp.float32),
        [pltpu.SemaphoreType.DMA] * 6,
        [pltpu.SemaphoreType.DMA] * 3,
        [pltpu.SemaphoreType.DMA] * 3,
    ],
)
def _sc_edge_kernel(table, idx, dst, norm, out,
                    idx3, dst3, nm3, rows3, acc, sem_s, sem_g, sem_sc):
    _sc_edge_body(table, idx, dst, norm, out,
                  idx3, dst3, nm3, rows3, acc, sem_s, sem_g, sem_sc)


def _add_body(a_ref, b_ref, o_ref):
    o_ref[...] = a_ref[...] + b_ref[...]


def _combine(partials):
    bn = 2000
    return pl.pallas_call(
        _add_body,
        grid=(N // bn,),
        in_specs=[
            pl.BlockSpec((bn, D), lambda i: (i, 0)),
            pl.BlockSpec((bn, D), lambda i: (i, 0)),
        ],
        out_specs=pl.BlockSpec((bn, D), lambda i: (i, 0)),
        out_shape=jax.ShapeDtypeStruct((N, D), jnp.float32),
    )(partials[:N], partials[N_PAD:N_PAD + N])


def kernel(x, edge_index, rel_type, norm, weight):
    src = edge_index[0].astype(jnp.int32)
    dst = edge_index[1].astype(jnp.int32)
    rel = rel_type.astype(jnp.int32)
    idx = rel * N + src

    pad = E_PAD - E
    idx_p = jnp.concatenate([idx, jnp.zeros((pad,), jnp.int32)])
    dst_p = jnp.concatenate([dst, jnp.zeros((pad,), jnp.int32)])
    norm_p = jnp.concatenate([norm[:, 0], jnp.zeros((pad,), jnp.float32)])

    table = _transform(x, weight)
    partials = _sc_edge_kernel(table, idx_p, dst_p, norm_p)
    return _combine(partials)


# speedup vs baseline: 22.7740x; 1.0206x over previous
import functools

import jax
import jax.numpy as jnp
from jax import lax
from jax.experimental import pallas as pl
from jax.experimental.pallas import tpu as pltpu
from jax.experimental.pallas import tpu_sc as plsc

N = 10000
E = 320000
D = 128
R = 8

NC = 2
NS = 16
NW = NC * NS
LANES = 16

CHUNK = 112
NCH = 90
NCH0 = 132
NCH1 = 2 * NCH - NCH0
UNROLL = 6
E_PAD = NW * CHUNK * NCH

N_PAD = 10240
OROWS = N_PAD // NS


def _matmul_body(x_ref, w_ref, o_ref):
    xb = x_ref[...]
    for r in range(R):
        o_ref[r] = jnp.dot(xb, w_ref[r], preferred_element_type=jnp.float32)


def _transform(x, weight):
    bn = 2000
    nb = N // bn
    out = pl.pallas_call(
        _matmul_body,
        grid=(nb,),
        in_specs=[
            pl.BlockSpec((bn, D), lambda n: (n, 0)),
            pl.BlockSpec((R, D, D), lambda n: (0, 0, 0)),
        ],
        out_specs=pl.BlockSpec((R, bn, D), lambda n: (0, n, 0)),
        out_shape=jax.ShapeDtypeStruct((R, N, D), jnp.float32),
    )(x, weight)
    return out.reshape(R * N, D)


def _sc_edge_body(table, idx, dst, norm, out,
                  idx3, dst3, nm3, rows3, acc, sem_s, sem_g, sem_sc):
    cid = lax.axis_index("c")
    sid = lax.axis_index("s")
    cbase = sid * (2 * NCH) + jnp.where(cid == 0, 0, NCH0)
    my_nch = jnp.where(cid == 0, NCH0, NCH1)

    zero16 = jnp.zeros((LANES,), jnp.float32)

    def _zero_row(e, _):
        for j in range(D // LANES):
            rows3[0, e, pl.ds(j * LANES, LANES)] = zero16
        return _

    lax.fori_loop(0, CHUNK, _zero_row, None)
    zoff = 0
    for zr in (CHUNK, CHUNK, CHUNK, CHUNK, CHUNK, OROWS - 5 * CHUNK):
        pltpu.sync_copy(rows3.at[0, pl.ds(0, zr)],
                        acc.at[pl.ds(sid * OROWS + zoff, zr)])
        zoff += zr
    plsc.subcore_barrier()

    def _small(c, s):
        base = (cbase + c) * CHUNK
        pltpu.async_copy(idx.at[pl.ds(base, CHUNK)], idx3.at[s], sem_s[s])
        pltpu.async_copy(dst.at[pl.ds(base, CHUNK)], dst3.at[s], sem_s[s])
        pltpu.async_copy(norm.at[pl.ds(base, CHUNK)], nm3.at[s], sem_s[s])

    def _wait_small(c, s):
        base = (cbase + c) * CHUNK
        pltpu.make_async_copy(idx.at[pl.ds(base, CHUNK)], idx3.at[s],
                              sem_s[s]).wait()
        pltpu.make_async_copy(dst.at[pl.ds(base, CHUNK)], dst3.at[s],
                              sem_s[s]).wait()
        pltpu.make_async_copy(norm.at[pl.ds(base, CHUNK)], nm3.at[s],
                              sem_s[s]).wait()

    def _gather(s, rb):
        pltpu.async_copy(table.at[idx3.at[s]], rows3.at[rb], sem_g[rb])

    def _wait_gather(s, rb):
        pltpu.make_async_copy(table.at[idx3.at[s]], rows3.at[rb],
                              sem_g[rb]).wait()

    def _scatter(s, rb):
        pltpu.async_copy(rows3.at[rb], acc.at[dst3.at[s]], sem_sc[rb],
                         add=True)

    def _wait_scatter(s, rb):
        pltpu.make_async_copy(rows3.at[rb], acc.at[dst3.at[s]],
                              sem_sc[rb]).wait()

    _small(0, 0)
    _small(1, 1)
    _wait_small(0, 0)
    _gather(0, 0)

    nblk = my_nch // UNROLL

    def _outer(i, _):
        for k in range(UNROLL):
            c = i * UNROLL + k
            s_cur = k % 6
            s_nxt = (k + 1) % 6
            s_n2 = (k + 2) % 6
            rb = k % 3
            rb_nxt = (k + 1) % 3
            last_blk = i == nblk - 1

            def _steady(pred_sc):
                if pred_sc is None:
                    _wait_scatter((k - 2) % 6, rb_nxt)
                else:
                    @pl.when(pred_sc)
                    def _():
                        _wait_scatter((k - 2) % 6, rb_nxt)

            if k == UNROLL - 1:
                @pl.when(jnp.logical_not(last_blk))
                def _():
                    _wait_small(c + 1, s_nxt)
                    _wait_scatter((k - 2) % 6, rb_nxt)
                    _gather(s_nxt, rb_nxt)
                    _small(c + 2, s_n2)

                @pl.when(last_blk)
                def _():
                    _wait_scatter((k - 2) % 6, rb_nxt)
            else:
                _wait_small(c + 1, s_nxt)
                _steady(i > 0 if k < 2 else None)
                _gather(s_nxt, rb_nxt)
                if k == UNROLL - 2:
                    @pl.when(jnp.logical_not(last_blk))
                    def _():
                        _small(c + 2, s_n2)
                else:
                    _small(c + 2, s_n2)

            _wait_gather(s_cur, rb)

            def _scale(g, carry):
                nv16 = nm3[s_cur, pl.ds(g * LANES, LANES)]
                for e in range(LANES):
                    row = g * LANES + e
                    nv = jnp.full((LANES,), nv16[e], jnp.float32)
                    for j in range(D // LANES):
                        sl = pl.ds(j * LANES, LANES)
                        rows3[rb, row, sl] = rows3[rb, row, sl] * nv
                return carry

            lax.fori_loop(0, CHUNK // LANES, _scale, None)
            _scatter(s_cur, rb)
        return _

    lax.fori_loop(0, nblk, _outer, None)
    _wait_scatter(4, 1)
    _wait_scatter(5, 2)
    plsc.subcore_barrier()

    pltpu.sync_copy(acc.at[pl.ds(sid * OROWS, OROWS)],
                    out.at[pl.ds(cid * N_PAD + sid * OROWS, OROWS)])


@functools.partial(
    pl.kernel,
    mesh=plsc.VectorSubcoreMesh(core_axis_name="c", subcore_axis_name="s",
                                num_cores=NC, num_subcores=NS),
    out_type=jax.ShapeDtypeStruct((NC * N_PAD, D), jnp.float32),
    scratch_types=[
        pltpu.VMEM((6, CHUNK), jnp.int32),
        pltpu.VMEM((6, CHUNK), jnp.int32),
        pltpu.VMEM((6, CHUNK), jnp.float32),
        pltpu.VMEM((3, CHUNK, D), jnp.float32),
        pltpu.VMEM_SHARED((N_PAD, D), jnp.float32),
        [pltpu.SemaphoreType.DMA] * 6,
        [pltpu.SemaphoreType.DMA] * 3,
        [pltpu.SemaphoreType.DMA] * 3,
    ],
)
def _sc_edge_kernel(table, idx, dst, norm, out,
                    idx3, dst3, nm3, rows3, acc, sem_s, sem_g, sem_sc):
    _sc_edge_body(table, idx, dst, norm, out,
                  idx3, dst3, nm3, rows3, acc, sem_s, sem_g, sem_sc)


def _add_body(a_ref, b_ref, o_ref):
    o_ref[...] = a_ref[...] + b_ref[...]


def _combine(partials):
    bn = 2000
    return pl.pallas_call(
        _add_body,
        grid=(N // bn,),
        in_specs=[
            pl.BlockSpec((bn, D), lambda i: (i, 0)),
            pl.BlockSpec((bn, D), lambda i: (i, 0)),
        ],
        out_specs=pl.BlockSpec((bn, D), lambda i: (i, 0)),
        out_shape=jax.ShapeDtypeStruct((N, D), jnp.float32),
    )(partials[:N], partials[N_PAD:N_PAD + N])


def kernel(x, edge_index, rel_type, norm, weight):
    src = edge_index[0].astype(jnp.int32)
    dst = edge_index[1].astype(jnp.int32)
    rel = rel_type.astype(jnp.int32)
    idx = rel * N + src

    pad = E_PAD - E
    idx_p = jnp.concatenate([idx, jnp.zeros((pad,), jnp.int32)])
    dst_p = jnp.concatenate([dst, jnp.zeros((pad,), jnp.int32)])
    norm_p = jnp.concatenate([norm[:, 0], jnp.zeros((pad,), jnp.float32)])

    table = _transform(x, weight)
    partials = _sc_edge_kernel(table, idx_p, dst_p, norm_p)
    return _combine(partials)
